# Initial kernel scaffold; baseline (speedup 1.0000x reference)
import jax
import jax.numpy as jnp
from jax.experimental import pallas as pl


def kernel(x, edge_index, share_state, value_batch, W1, b1, W2, b2, W3, b3, W4, b4, Wv1, bv1, Wv2, bv2, Wv3, bv3, Wc1, bc1, Wc2, bc2):
    raise NotImplementedError("write your pallas kernel here")


# speedup vs baseline: 25.9411x; 25.9411x over previous
import functools

import jax
import jax.numpy as jnp
from jax import lax
from jax.experimental import pallas as pl
from jax.experimental.pallas import tpu as pltpu
from jax.experimental.pallas import tpu_sc as plsc

N = 10000
E = 320000
B = 64
D = 128
H = 32

NP = 10240
ROWS_PER_TILE = NP // 16
NTILES = 32
CHUNK = 128
CHUNKS_PER_TILE = 80
EDGES_PER_TILE = CHUNK * CHUNKS_PER_TILE
EP = NTILES * EDGES_PER_TILE
RB = 1024
NBLK = NP // RB


def _mesh():
    return plsc.VectorSubcoreMesh(core_axis_name="c", subcore_axis_name="s")


def _sc_degree(dstp):
    @functools.partial(
        pl.kernel,
        out_type=jax.ShapeDtypeStruct((2, NP), jnp.float32),
        mesh=_mesh(),
        compiler_params=pltpu.CompilerParams(use_tc_tiling_on_sc=False),
        scratch_types=[
            pltpu.VMEM((CHUNKS_PER_TILE, CHUNK), jnp.int32),
            pltpu.VMEM((CHUNK,), jnp.float32),
            pltpu.VMEM((ROWS_PER_TILE,), jnp.float32),
            pltpu.VMEM_SHARED((NP,), jnp.float32),
        ],
    )
    def k(dstp_hbm, out_hbm, didx, ones_v, ibuf, dacc):
        c = lax.axis_index("c")
        s = lax.axis_index("s")
        wid = c * 16 + s
        pltpu.sync_copy(dstp_hbm.at[wid], didx)
        init = jnp.where(c == 0, 1.0, 0.0).astype(jnp.float32)

        def fill_i(r, carry):
            ibuf[pl.ds(r * 16, 16)] = jnp.zeros((16,), jnp.float32) + init
            return carry

        lax.fori_loop(0, ROWS_PER_TILE // 16, fill_i, 0)

        def fill_o(r, carry):
            ones_v[pl.ds(r * 16, 16)] = jnp.ones((16,), jnp.float32)
            return carry

        lax.fori_loop(0, CHUNK // 16, fill_o, 0)

        pltpu.sync_copy(ibuf, dacc.at[pl.ds(s * ROWS_PER_TILE, ROWS_PER_TILE)])
        plsc.subcore_barrier()

        def step(g, carry):
            pltpu.sync_copy(ones_v, dacc.at[didx.at[g]], add=True)
            return carry

        lax.fori_loop(0, CHUNKS_PER_TILE, step, 0)
        plsc.subcore_barrier()
        pltpu.sync_copy(dacc.at[pl.ds(s * ROWS_PER_TILE, ROWS_PER_TILE)], ibuf)
        pltpu.sync_copy(ibuf, out_hbm.at[c, pl.ds(s * ROWS_PER_TILE, ROWS_PER_TILE)])

    return k(dstp)


def _sc_edge_agg(y, srcp, dstp):
    @functools.partial(
        pl.kernel,
        out_type=jax.ShapeDtypeStruct((2, NP, H), jnp.float32),
        mesh=_mesh(),
        compiler_params=pltpu.CompilerParams(use_tc_tiling_on_sc=False),
        scratch_types=[
            pltpu.VMEM((CHUNKS_PER_TILE, CHUNK), jnp.int32),
            pltpu.VMEM((CHUNKS_PER_TILE, CHUNK), jnp.int32),
            pltpu.VMEM((CHUNK, H), jnp.float32),
            pltpu.VMEM((CHUNK, H), jnp.float32),
            pltpu.VMEM((ROWS_PER_TILE, H), jnp.float32),
            pltpu.VMEM_SHARED((NP, H), jnp.float32),
            pltpu.SemaphoreType.DMA,
            pltpu.SemaphoreType.DMA,
        ],
    )
    def k(y_hbm, srcp_hbm, dstp_hbm, out_hbm, sidx, didx, rows_a, rows_b,
          zbuf, acc, sem_a, sem_b):
        c = lax.axis_index("c")
        s = lax.axis_index("s")
        wid = c * 16 + s
        pltpu.sync_copy(srcp_hbm.at[wid], sidx)
        pltpu.sync_copy(dstp_hbm.at[wid], didx)

        def zr(r, carry):
            zbuf[r, pl.ds(0, 16)] = jnp.zeros((16,), jnp.float32)
            zbuf[r, pl.ds(16, 16)] = jnp.zeros((16,), jnp.float32)
            return carry

        lax.fori_loop(0, ROWS_PER_TILE, zr, 0)
        pltpu.sync_copy(zbuf, acc.at[pl.ds(s * ROWS_PER_TILE, ROWS_PER_TILE)])
        plsc.subcore_barrier()

        pltpu.async_copy(y_hbm.at[sidx.at[0]], rows_a, sem_a)

        def step(t, carry):
            g = 2 * t
            pltpu.async_copy(y_hbm.at[sidx.at[g + 1]], rows_b, sem_b)
            pltpu.make_async_copy(y_hbm.at[sidx.at[g]], rows_a, sem_a).wait()
            pltpu.sync_copy(rows_a, acc.at[didx.at[g]], add=True)

            @pl.when(t < CHUNKS_PER_TILE // 2 - 1)
            def _():
                pltpu.async_copy(y_hbm.at[sidx.at[g + 2]], rows_a, sem_a)

            pltpu.make_async_copy(y_hbm.at[sidx.at[g + 1]], rows_b, sem_b).wait()
            pltpu.sync_copy(rows_b, acc.at[didx.at[g + 1]], add=True)
            return carry

        lax.fori_loop(0, CHUNKS_PER_TILE // 2, step, 0)
        plsc.subcore_barrier()
        pltpu.sync_copy(acc.at[pl.ds(s * ROWS_PER_TILE, ROWS_PER_TILE)], zbuf)
        pltpu.sync_copy(zbuf, out_hbm.at[c, pl.ds(s * ROWS_PER_TILE, ROWS_PER_TILE)])

    return k(y, srcp, dstp)


def _tc_scale_mm(x_p, degp, W):
    def body(x_ref, d_ref, w_ref, o_ref):
        dinv = lax.rsqrt(d_ref[0] + d_ref[1])
        o_ref[...] = dinv * jnp.dot(x_ref[...], w_ref[...],
                                    preferred_element_type=jnp.float32)

    din = x_p.shape[1]
    return pl.pallas_call(
        body,
        grid=(NBLK,),
        in_specs=[
            pl.BlockSpec((RB, din), lambda i: (i, 0)),
            pl.BlockSpec((2, RB, 1), lambda i: (0, i, 0)),
            pl.BlockSpec((din, H), lambda i: (0, 0)),
        ],
        out_specs=pl.BlockSpec((RB, H), lambda i: (i, 0)),
        out_shape=jax.ShapeDtypeStruct((NP, H), jnp.float32),
    )(x_p, degp, W)


def _tc_layer_mid(accp, y1, degp, b1, W2):
    def body(a_ref, y_ref, d_ref, b_ref, w_ref, o_ref):
        dinv = lax.rsqrt(d_ref[0] + d_ref[1])
        agg = a_ref[0] + a_ref[1] + y_ref[...]
        h1 = jnp.tanh(dinv * agg + b_ref[...])
        o_ref[...] = dinv * jnp.dot(h1, w_ref[...],
                                    preferred_element_type=jnp.float32)

    return pl.pallas_call(
        body,
        grid=(NBLK,),
        in_specs=[
            pl.BlockSpec((2, RB, H), lambda i: (0, i, 0)),
            pl.BlockSpec((RB, H), lambda i: (i, 0)),
            pl.BlockSpec((2, RB, 1), lambda i: (0, i, 0)),
            pl.BlockSpec((1, H), lambda i: (0, 0)),
            pl.BlockSpec((H, H), lambda i: (0, 0)),
        ],
        out_specs=pl.BlockSpec((RB, H), lambda i: (i, 0)),
        out_shape=jax.ShapeDtypeStruct((NP, H), jnp.float32),
    )(accp, y1, degp, b1, W2)


def _tc_final(accp, y2, degp, vb, b2, W3, b3, W4, b4, ss,
              Wv1, bv1, Wv2, bv2, Wv3, bv3, Wc1, bc1, Wc2, bc2):
    def body(a_ref, y_ref, d_ref, vb_ref, b2_ref, w3_ref, b3_ref, w4_ref,
             b4_ref, ss_ref, wv1_ref, bv1_ref, wv2_ref, bv2_ref, wv3_ref,
             bv3_ref, wc1_ref, bc1_ref, wc2_ref, bc2_ref, o_ref, g_ref):
        i = pl.program_id(0)
        dinv = lax.rsqrt(d_ref[0] + d_ref[1])
        agg = a_ref[0] + a_ref[1] + y_ref[...]
        h2 = jnp.tanh(dinv * agg + b2_ref[...])
        h3 = jnp.tanh(jnp.dot(h2, w3_ref[...],
                              preferred_element_type=jnp.float32) + b3_ref[...])
        seg = lax.broadcasted_iota(jnp.int32, (1, B), 1)
        pb = (vb_ref[...] == seg).astype(jnp.float32)
        part = lax.dot_general(pb, h3, (((0,), (0,)), ((), ())),
                               preferred_element_type=jnp.float32)

        @pl.when(i == 0)
        def _():
            g_ref[...] = part

        @pl.when(i > 0)
        def _():
            g_ref[...] = g_ref[...] + part

        @pl.when(i == NBLK - 1)
        def _():
            g = g_ref[...]
            h1v = jnp.dot(g, w4_ref[...],
                          preferred_element_type=jnp.float32) + b4_ref[...]
            t = jnp.tanh(jnp.dot(ss_ref[...], wv1_ref[...],
                                 preferred_element_type=jnp.float32) + bv1_ref[...])
            t = jnp.tanh(jnp.dot(t, wv2_ref[...],
                                 preferred_element_type=jnp.float32) + bv2_ref[...])
            h2v = jnp.dot(t, wv3_ref[...],
                          preferred_element_type=jnp.float32) + bv3_ref[...]
            cc = jnp.tanh(jnp.dot(h1v, wc1_ref[pl.ds(0, H), :],
                                  preferred_element_type=jnp.float32)
                          + jnp.dot(h2v, wc1_ref[pl.ds(H, H), :],
                                    preferred_element_type=jnp.float32)
                          + bc1_ref[...])
            o_ref[...] = jnp.dot(cc, wc2_ref[...],
                                 preferred_element_type=jnp.float32) + bc2_ref[...]

    full = lambda shp: pl.BlockSpec(shp, lambda i: tuple(0 for _ in shp))
    return pl.pallas_call(
        body,
        grid=(NBLK,),
        in_specs=[
            pl.BlockSpec((2, RB, H), lambda i: (0, i, 0)),
            pl.BlockSpec((RB, H), lambda i: (i, 0)),
            pl.BlockSpec((2, RB, 1), lambda i: (0, i, 0)),
            pl.BlockSpec((RB, 1), lambda i: (i, 0)),
            full((1, H)), full((H, H)), full((1, H)), full((H, H)),
            full((1, H)), full((B, D)), full((D, H)), full((1, H)),
            full((H, H)), full((1, H)), full((H, H)), full((1, H)),
            full((2 * H, H)), full((1, H)), full((H, 1)), full((1, 1)),
        ],
        out_specs=pl.BlockSpec((B, 1), lambda i: (0, 0)),
        out_shape=jax.ShapeDtypeStruct((B, 1), jnp.float32),
        scratch_shapes=[pltpu.VMEM((B, H), jnp.float32)],
    )(accp, y2, degp, vb, b2, W3, b3, W4, b4, ss,
      Wv1, bv1, Wv2, bv2, Wv3, bv3, Wc1, bc1, Wc2, bc2)


def kernel(x, edge_index, share_state, value_batch, W1, b1, W2, b2, W3, b3,
           W4, b4, Wv1, bv1, Wv2, bv2, Wv3, bv3, Wc1, bc1, Wc2, bc2):
    f32 = jnp.float32
    i32 = jnp.int32

    x_p = jnp.concatenate([x, jnp.zeros((NP - N, D), f32)], axis=0)
    src = edge_index[0].astype(i32)
    dst = edge_index[1].astype(i32)
    srcp = jnp.concatenate([src, jnp.zeros((EP - E,), i32)]).reshape(
        NTILES, CHUNKS_PER_TILE, CHUNK)
    dstp = jnp.concatenate([dst, jnp.full((EP - E,), N, i32)]).reshape(
        NTILES, CHUNKS_PER_TILE, CHUNK)
    vb = jnp.concatenate([value_batch.astype(i32),
                          jnp.full((NP - N,), B, i32)]).reshape(NP, 1)

    degp = _sc_degree(dstp).reshape(2, NP, 1)

    y1 = _tc_scale_mm(x_p, degp, W1)
    acc1 = _sc_edge_agg(y1, srcp, dstp)
    y2 = _tc_layer_mid(acc1, y1, degp, b1.reshape(1, H), W2)
    acc2 = _sc_edge_agg(y2, srcp, dstp)
    value = _tc_final(acc2, y2, degp, vb, b2.reshape(1, H), W3,
                      b3.reshape(1, H), W4, b4.reshape(1, H), share_state,
                      Wv1, bv1.reshape(1, H), Wv2, bv2.reshape(1, H), Wv3,
                      bv3.reshape(1, H), Wc1, bc1.reshape(1, H), Wc2,
                      bc2.reshape(1, 1))
    return value
